# baseline (speedup 1.0000x reference)
import functools

import jax
import jax.numpy as jnp
from jax import lax
from jax.experimental import pallas as pl
from jax.experimental.pallas import tpu as pltpu
from jax.experimental.pallas import tpu_sc as plsc

_N = 10000
_NPAD = 10240
_RPT = 640
_PADROW = 10200
_E = 320000
_CHUNK = 128
_NCHUNK = 80
_EPT = _NCHUNK * _CHUNK
_EPAD = _EPT * 32

@functools.cache
def _make_deg_kernel():
  _mesh = plsc.VectorSubcoreMesh(core_axis_name="c", subcore_axis_name="s")
  @functools.partial(
      pl.kernel, mesh=_mesh,
      out_type=jax.ShapeDtypeStruct((2, _NPAD), jnp.float32),
      scratch_types=[
          pltpu.VMEM((_CHUNK,), jnp.int32),
          pltpu.VMEM((_CHUNK,), jnp.float32),
          pltpu.VMEM((_RPT,), jnp.float32),
          pltpu.VMEM_SHARED((_NPAD,), jnp.float32),
      ],
  )
  def k(dst_hbm, ones_hbm, zeros_hbm, out_hbm, didx, onesv, stage, acc):
    cid = lax.axis_index("c")
    sid = lax.axis_index("s")
    wid = cid * 16 + sid
    base_r = sid * _RPT
    pltpu.sync_copy(ones_hbm, onesv)
    pltpu.sync_copy(zeros_hbm, stage)
    pltpu.sync_copy(stage, acc.at[pl.ds(base_r, _RPT)])
    plsc.subcore_barrier()

    def body(i, carry):
      pltpu.sync_copy(dst_hbm.at[wid * _NCHUNK + i], didx)
      pltpu.sync_copy(onesv, acc.at[didx], add=True)
      return carry

    lax.fori_loop(0, _NCHUNK, body, 0)
    plsc.subcore_barrier()
    pltpu.sync_copy(acc.at[pl.ds(base_r, _RPT)], stage)
    pltpu.sync_copy(stage, out_hbm.at[cid, pl.ds(base_r, _RPT)])

  return k


@functools.cache
def _make_scatter_kernel(d):
  _mesh = plsc.VectorSubcoreMesh(core_axis_name="c", subcore_axis_name="s")
  @functools.partial(
      pl.kernel, mesh=_mesh,
      compiler_params=pltpu.CompilerParams(use_tc_tiling_on_sc=(d == 128)),
      out_type=jax.ShapeDtypeStruct((2, _NPAD, d), jnp.float32),
      scratch_types=[
          pltpu.VMEM((_NCHUNK, _CHUNK), jnp.int32),
          pltpu.VMEM((_CHUNK,), jnp.int32),
          pltpu.VMEM((_CHUNK,), jnp.int32),
          pltpu.VMEM((_CHUNK, d), jnp.float32),
          pltpu.VMEM((_CHUNK, d), jnp.float32),
          pltpu.VMEM_SHARED((_NPAD, d), jnp.float32),
          pltpu.SemaphoreType.DMA,
          pltpu.SemaphoreType.DMA,
      ],
  )
  def k(h_hbm, src_hbm, dst_hbm, zeros_hbm, out_hbm, sidx, didxa, didxb,
        bufa, bufb, acc, sema, semb):
    cid = lax.axis_index("c")
    sid = lax.axis_index("s")
    wid = cid * 16 + sid
    base_r = sid * _RPT
    cbase = wid * _NCHUNK
    pltpu.sync_copy(src_hbm.at[pl.ds(cbase, _NCHUNK)], sidx)
    pltpu.sync_copy(zeros_hbm, bufa)

    def zbody(j, carry):
      pltpu.sync_copy(bufa, acc.at[pl.ds(base_r + j * _CHUNK, _CHUNK)])
      return carry

    lax.fori_loop(0, _RPT // _CHUNK, zbody, 0)
    plsc.subcore_barrier()

    pltpu.async_copy(h_hbm.at[sidx.at[0]], bufa, sema)
    pltpu.sync_copy(dst_hbm.at[cbase], didxa)

    def body(i, carry):
      c0 = 2 * i
      pltpu.async_copy(h_hbm.at[sidx.at[c0 + 1]], bufb, semb)
      pltpu.sync_copy(dst_hbm.at[cbase + c0 + 1], didxb)
      pltpu.make_async_copy(h_hbm.at[sidx.at[c0]], bufa, sema).wait()
      pltpu.sync_copy(bufa, acc.at[didxa], add=True)

      @pl.when(i < _NCHUNK // 2 - 1)
      def _():
        pltpu.async_copy(h_hbm.at[sidx.at[c0 + 2]], bufa, sema)
        pltpu.sync_copy(dst_hbm.at[cbase + c0 + 2], didxa)

      pltpu.make_async_copy(h_hbm.at[sidx.at[c0 + 1]], bufb, semb).wait()
      pltpu.sync_copy(bufb, acc.at[didxb], add=True)
      return carry

    lax.fori_loop(0, _NCHUNK // 2, body, 0)
    plsc.subcore_barrier()

    def wbody(j, carry):
      r0 = base_r + j * _CHUNK
      pltpu.sync_copy(acc.at[pl.ds(r0, _CHUNK)], bufa)
      pltpu.sync_copy(bufa, out_hbm.at[cid, pl.ds(r0, _CHUNK)])
      return carry

    lax.fori_loop(0, _RPT // _CHUNK, wbody, 0)

  return k


_BLK = 1024
_GRID = _NPAD // _BLK


def _tc_a(x_ref, w_ref, deg_ref, h_ref, dinv_ref):
  deg = deg_ref[0] + deg_ref[1] + 1.0
  dinv = lax.rsqrt(deg)
  h = jnp.dot(x_ref[...], w_ref[...], preferred_element_type=jnp.float32)
  h_ref[...] = h * dinv[:, None]
  dinv_ref[...] = dinv


def _tc_b(acc_ref, h1_ref, dinv_ref, b1_ref, w2_ref, h2_ref):
  dinv = dinv_ref[...]
  s = acc_ref[0] + acc_ref[1] + h1_ref[...]
  t = jnp.maximum(s * dinv[:, None] + b1_ref[...], 0.0)
  h2 = jnp.dot(t, w2_ref[...], preferred_element_type=jnp.float32)
  h2_ref[...] = h2 * dinv[:, None]


def _tc_c(acc_ref, h2_ref, dinv_ref, b2_ref, o_ref):
  dinv = dinv_ref[...]
  s = acc_ref[0] + acc_ref[1] + h2_ref[...]
  o_ref[...] = s * dinv[:, None] + b2_ref[...]


def _stage_a(x_pad, W1, deg2):
  return pl.pallas_call(
      _tc_a,
      grid=(_GRID,),
      in_specs=[
          pl.BlockSpec((_BLK, 128), lambda i: (i, 0)),
          pl.BlockSpec((128, 128), lambda i: (0, 0)),
          pl.BlockSpec((2, _BLK), lambda i: (0, i)),
      ],
      out_specs=[
          pl.BlockSpec((_BLK, 128), lambda i: (i, 0)),
          pl.BlockSpec((_BLK,), lambda i: (i,)),
      ],
      out_shape=[
          jax.ShapeDtypeStruct((_NPAD, 128), jnp.float32),
          jax.ShapeDtypeStruct((_NPAD,), jnp.float32),
      ],
  )(x_pad, W1, deg2)


def _stage_b(acc1, h1p, dinv, b1, W2):
  return pl.pallas_call(
      _tc_b,
      grid=(_GRID,),
      in_specs=[
          pl.BlockSpec((2, _BLK, 128), lambda i: (0, i, 0)),
          pl.BlockSpec((_BLK, 128), lambda i: (i, 0)),
          pl.BlockSpec((_BLK,), lambda i: (i,)),
          pl.BlockSpec((128,), lambda i: (0,)),
          pl.BlockSpec((128, 64), lambda i: (0, 0)),
      ],
      out_specs=pl.BlockSpec((_BLK, 64), lambda i: (i, 0)),
      out_shape=jax.ShapeDtypeStruct((_NPAD, 64), jnp.float32),
  )(acc1, h1p, dinv, b1, W2)


def _stage_c(acc2, h2p, dinv, b2):
  return pl.pallas_call(
      _tc_c,
      grid=(_GRID,),
      in_specs=[
          pl.BlockSpec((2, _BLK, 64), lambda i: (0, i, 0)),
          pl.BlockSpec((_BLK, 64), lambda i: (i, 0)),
          pl.BlockSpec((_BLK,), lambda i: (i,)),
          pl.BlockSpec((64,), lambda i: (0,)),
      ],
      out_specs=pl.BlockSpec((_BLK, 64), lambda i: (i, 0)),
      out_shape=jax.ShapeDtypeStruct((_NPAD, 64), jnp.float32),
  )(acc2, h2p, dinv, b2)


def kernel(x, edge_index, W1, b1, W2, b2):
  src = edge_index[0].astype(jnp.int32)
  dst = edge_index[1].astype(jnp.int32)
  pad = jnp.full((_EPAD - _E,), _PADROW, dtype=jnp.int32)
  src_p = jnp.concatenate([src, pad]).reshape(_EPAD // _CHUNK, _CHUNK)
  dst_p = jnp.concatenate([dst, pad]).reshape(_EPAD // _CHUNK, _CHUNK)

  ones_c = jnp.ones((_CHUNK,), jnp.float32)
  zeros_r = jnp.zeros((_RPT,), jnp.float32)
  zeros128 = jnp.zeros((_CHUNK, 128), jnp.float32)
  zeros64 = jnp.zeros((_CHUNK, 64), jnp.float32)
  x_pad = jnp.zeros((_NPAD, 128), jnp.float32).at[:_N].set(x)

  deg2 = _make_deg_kernel()(dst_p, ones_c, zeros_r)
  h1p, dinv = _stage_a(x_pad, W1, deg2)
  acc1 = _make_scatter_kernel(128)(h1p, src_p, dst_p, zeros128)
  h2p = _stage_b(acc1, h1p, dinv, b1, W2)
  acc2 = _make_scatter_kernel(64)(h2p, src_p, dst_p, zeros64)
  out = _stage_c(acc2, h2p, dinv, b2)
  return out[:_N]


# speedup vs baseline: 2.5710x; 2.5710x over previous
import functools

import jax
import jax.numpy as jnp
from jax import lax
from jax.experimental import pallas as pl
from jax.experimental.pallas import tpu as pltpu
from jax.experimental.pallas import tpu_sc as plsc

_N = 10000
_NPAD = 10240
_RPT = 640
_E = 320000
_CHUNK = 128
_NCHUNK = 80
_EPT = _NCHUNK * _CHUNK
_EPAD = _EPT * 32

@functools.cache
def _make_deg_kernel():
  _mesh = plsc.VectorSubcoreMesh(core_axis_name="c", subcore_axis_name="s")
  @functools.partial(
      pl.kernel, mesh=_mesh,
      out_type=jax.ShapeDtypeStruct((2, _NPAD), jnp.float32),
      scratch_types=[
          pltpu.VMEM((_CHUNK,), jnp.int32),
          pltpu.VMEM((_CHUNK,), jnp.float32),
          pltpu.VMEM((_RPT,), jnp.float32),
          pltpu.VMEM_SHARED((_NPAD,), jnp.float32),
      ],
  )
  def k(dst_hbm, ones_hbm, zeros_hbm, out_hbm, didx, onesv, stage, acc):
    cid = lax.axis_index("c")
    sid = lax.axis_index("s")
    wid = cid * 16 + sid
    base_r = sid * _RPT
    pltpu.sync_copy(ones_hbm, onesv)
    pltpu.sync_copy(zeros_hbm, stage)
    pltpu.sync_copy(stage, acc.at[pl.ds(base_r, _RPT)])
    plsc.subcore_barrier()

    def body(i, carry):
      pltpu.sync_copy(dst_hbm.at[wid * _NCHUNK + i], didx)
      pltpu.sync_copy(onesv, acc.at[didx], add=True)
      return carry

    lax.fori_loop(0, _NCHUNK, body, 0)
    plsc.subcore_barrier()
    pltpu.sync_copy(acc.at[pl.ds(base_r, _RPT)], stage)
    pltpu.sync_copy(stage, out_hbm.at[cid, pl.ds(base_r, _RPT)])

  return k


@functools.cache
def _make_scatter_kernel(d):
  _mesh = plsc.VectorSubcoreMesh(core_axis_name="c", subcore_axis_name="s")
  @functools.partial(
      pl.kernel, mesh=_mesh,
      compiler_params=pltpu.CompilerParams(use_tc_tiling_on_sc=(d == 128)),
      out_type=jax.ShapeDtypeStruct((2, _NPAD, d), jnp.float32),
      scratch_types=[
          pltpu.VMEM((_NCHUNK, _CHUNK), jnp.int32),
          pltpu.VMEM((_CHUNK,), jnp.int32),
          pltpu.VMEM((_CHUNK,), jnp.int32),
          pltpu.VMEM((_CHUNK, d), jnp.float32),
          pltpu.VMEM((_CHUNK, d), jnp.float32),
          pltpu.VMEM_SHARED((_NPAD, d), jnp.float32),
          pltpu.SemaphoreType.DMA,
          pltpu.SemaphoreType.DMA,
      ],
  )
  def k(h_hbm, src_hbm, dst_hbm, zeros_hbm, out_hbm, sidx, didxa, didxb,
        bufa, bufb, acc, sema, semb):
    cid = lax.axis_index("c")
    sid = lax.axis_index("s")
    wid = cid * 16 + sid
    base_r = sid * _RPT
    cbase = wid * _NCHUNK
    pltpu.sync_copy(src_hbm.at[pl.ds(cbase, _NCHUNK)], sidx)
    pltpu.sync_copy(zeros_hbm, bufa)

    def zbody(j, carry):
      pltpu.sync_copy(bufa, acc.at[pl.ds(base_r + j * _CHUNK, _CHUNK)])
      return carry

    lax.fori_loop(0, _RPT // _CHUNK, zbody, 0)
    plsc.subcore_barrier()

    pltpu.async_copy(h_hbm.at[sidx.at[0]], bufa, sema)
    pltpu.sync_copy(dst_hbm.at[cbase], didxa)

    def body(i, carry):
      c0 = 2 * i
      pltpu.async_copy(h_hbm.at[sidx.at[c0 + 1]], bufb, semb)
      pltpu.sync_copy(dst_hbm.at[cbase + c0 + 1], didxb)
      pltpu.make_async_copy(h_hbm.at[sidx.at[c0]], bufa, sema).wait()
      pltpu.sync_copy(bufa, acc.at[didxa], add=True)

      @pl.when(i < _NCHUNK // 2 - 1)
      def _():
        pltpu.async_copy(h_hbm.at[sidx.at[c0 + 2]], bufa, sema)
        pltpu.sync_copy(dst_hbm.at[cbase + c0 + 2], didxa)

      pltpu.make_async_copy(h_hbm.at[sidx.at[c0 + 1]], bufb, semb).wait()
      pltpu.sync_copy(bufb, acc.at[didxb], add=True)
      return carry

    lax.fori_loop(0, _NCHUNK // 2, body, 0)
    plsc.subcore_barrier()

    def wbody(j, carry):
      r0 = base_r + j * _CHUNK
      pltpu.sync_copy(acc.at[pl.ds(r0, _CHUNK)], bufa)
      pltpu.sync_copy(bufa, out_hbm.at[cid, pl.ds(r0, _CHUNK)])
      return carry

    lax.fori_loop(0, _RPT // _CHUNK, wbody, 0)

  return k


_BLK = 1024
_GRID = _NPAD // _BLK


def _tc_a(x_ref, w_ref, deg_ref, h_ref, dinv_ref):
  deg = deg_ref[0] + deg_ref[1] + 1.0
  dinv = lax.rsqrt(deg)
  h = jnp.dot(x_ref[...], w_ref[...], preferred_element_type=jnp.float32)
  h_ref[...] = h * dinv[:, None]
  dinv_ref[...] = dinv


def _tc_b(acc_ref, h1_ref, dinv_ref, b1_ref, w2_ref, h2_ref):
  dinv = dinv_ref[...]
  s = acc_ref[0] + acc_ref[1] + h1_ref[...]
  t = jnp.maximum(s * dinv[:, None] + b1_ref[...], 0.0)
  h2 = jnp.dot(t, w2_ref[...], preferred_element_type=jnp.float32)
  h2_ref[...] = h2 * dinv[:, None]


def _tc_c(acc_ref, h2_ref, dinv_ref, b2_ref, o_ref):
  dinv = dinv_ref[...]
  s = acc_ref[0] + acc_ref[1] + h2_ref[...]
  o_ref[...] = s * dinv[:, None] + b2_ref[...]


def _stage_a(x_pad, W1, deg2):
  return pl.pallas_call(
      _tc_a,
      grid=(_GRID,),
      in_specs=[
          pl.BlockSpec((_BLK, 128), lambda i: (i, 0)),
          pl.BlockSpec((128, 128), lambda i: (0, 0)),
          pl.BlockSpec((2, _BLK), lambda i: (0, i)),
      ],
      out_specs=[
          pl.BlockSpec((_BLK, 128), lambda i: (i, 0)),
          pl.BlockSpec((_BLK,), lambda i: (i,)),
      ],
      out_shape=[
          jax.ShapeDtypeStruct((_NPAD, 128), jnp.float32),
          jax.ShapeDtypeStruct((_NPAD,), jnp.float32),
      ],
  )(x_pad, W1, deg2)


def _stage_b(acc1, h1p, dinv, b1, W2):
  return pl.pallas_call(
      _tc_b,
      grid=(_GRID,),
      in_specs=[
          pl.BlockSpec((2, _BLK, 128), lambda i: (0, i, 0)),
          pl.BlockSpec((_BLK, 128), lambda i: (i, 0)),
          pl.BlockSpec((_BLK,), lambda i: (i,)),
          pl.BlockSpec((128,), lambda i: (0,)),
          pl.BlockSpec((128, 64), lambda i: (0, 0)),
      ],
      out_specs=pl.BlockSpec((_BLK, 64), lambda i: (i, 0)),
      out_shape=jax.ShapeDtypeStruct((_NPAD, 64), jnp.float32),
  )(acc1, h1p, dinv, b1, W2)


def _stage_c(acc2, h2p, dinv, b2):
  return pl.pallas_call(
      _tc_c,
      grid=(_GRID,),
      in_specs=[
          pl.BlockSpec((2, _BLK, 64), lambda i: (0, i, 0)),
          pl.BlockSpec((_BLK, 64), lambda i: (i, 0)),
          pl.BlockSpec((_BLK,), lambda i: (i,)),
          pl.BlockSpec((64,), lambda i: (0,)),
      ],
      out_specs=pl.BlockSpec((_BLK, 64), lambda i: (i, 0)),
      out_shape=jax.ShapeDtypeStruct((_NPAD, 64), jnp.float32),
  )(acc2, h2p, dinv, b2)


def kernel(x, edge_index, W1, b1, W2, b2):
  src = edge_index[0].astype(jnp.int32)
  dst = edge_index[1].astype(jnp.int32)
  ppt = (_EPAD - _E) // 32
  padblk = jnp.broadcast_to(_N + jnp.arange(ppt, dtype=jnp.int32), (32, ppt))

  def _pad_edges(a):
    a2 = jnp.concatenate([a.reshape(32, _E // 32), padblk], axis=1)
    return a2.reshape(_EPAD // _CHUNK, _CHUNK)

  src_p = _pad_edges(src)
  dst_p = _pad_edges(dst)

  ones_c = jnp.ones((_CHUNK,), jnp.float32)
  zeros_r = jnp.zeros((_RPT,), jnp.float32)
  zeros128 = jnp.zeros((_CHUNK, 128), jnp.float32)
  zeros64 = jnp.zeros((_CHUNK, 64), jnp.float32)
  x_pad = jnp.zeros((_NPAD, 128), jnp.float32).at[:_N].set(x)

  deg2 = _make_deg_kernel()(dst_p, ones_c, zeros_r)
  h1p, dinv = _stage_a(x_pad, W1, deg2)
  acc1 = _make_scatter_kernel(128)(h1p, src_p, dst_p, zeros128)
  h2p = _stage_b(acc1, h1p, dinv, b1, W2)
  acc2 = _make_scatter_kernel(64)(h2p, src_p, dst_p, zeros64)
  out = _stage_c(acc2, h2p, dinv, b2)
  return out[:_N]


# speedup vs baseline: 2.6001x; 1.0113x over previous
import functools

import jax
import jax.numpy as jnp
from jax import lax
from jax.experimental import pallas as pl
from jax.experimental.pallas import tpu as pltpu
from jax.experimental.pallas import tpu_sc as plsc

_N = 10000
_NPAD = 10240
_RPT = 640
_E = 320000
_CHUNK = 128
_NCHUNK = 80
_EPT = _NCHUNK * _CHUNK
_EPAD = _EPT * 32

@functools.cache
def _make_deg_kernel():
  _mesh = plsc.VectorSubcoreMesh(core_axis_name="c", subcore_axis_name="s")
  @functools.partial(
      pl.kernel, mesh=_mesh,
      out_type=jax.ShapeDtypeStruct((2, _NPAD), jnp.float32),
      scratch_types=[
          pltpu.VMEM((_NCHUNK, _CHUNK), jnp.int32),
          pltpu.VMEM((_CHUNK,), jnp.float32),
          pltpu.VMEM((_RPT,), jnp.float32),
          pltpu.VMEM_SHARED((_NPAD,), jnp.float32),
          pltpu.SemaphoreType.DMA,
      ],
  )
  def k(dst_hbm, ones_hbm, zeros_hbm, out_hbm, didx, onesv, stage, acc, sem):
    cid = lax.axis_index("c")
    sid = lax.axis_index("s")
    wid = cid * 16 + sid
    base_r = sid * _RPT
    pltpu.sync_copy(dst_hbm.at[pl.ds(wid * _NCHUNK, _NCHUNK)], didx)
    pltpu.sync_copy(ones_hbm, onesv)
    pltpu.sync_copy(zeros_hbm, stage)
    pltpu.sync_copy(stage, acc.at[pl.ds(base_r, _RPT)])
    plsc.subcore_barrier()

    def body(i, carry):
      for j in range(8):
        pltpu.async_copy(onesv, acc.at[didx.at[i * 8 + j]], sem, add=True)
      for j in range(8):
        pltpu.make_async_copy(onesv, acc.at[didx.at[i * 8 + j]], sem).wait()
      return carry

    lax.fori_loop(0, _NCHUNK // 8, body, 0)
    plsc.subcore_barrier()
    pltpu.sync_copy(acc.at[pl.ds(base_r, _RPT)], stage)
    pltpu.sync_copy(stage, out_hbm.at[cid, pl.ds(base_r, _RPT)])

  return k


@functools.cache
def _make_scatter_kernel(d):
  _mesh = plsc.VectorSubcoreMesh(core_axis_name="c", subcore_axis_name="s")
  @functools.partial(
      pl.kernel, mesh=_mesh,
      compiler_params=pltpu.CompilerParams(use_tc_tiling_on_sc=(d == 128)),
      out_type=jax.ShapeDtypeStruct((2, _NPAD, d), jnp.float32),
      scratch_types=[
          pltpu.VMEM((_NCHUNK, _CHUNK), jnp.int32),
          pltpu.VMEM((_CHUNK,), jnp.int32),
          pltpu.VMEM((_CHUNK,), jnp.int32),
          pltpu.VMEM((_CHUNK, d), jnp.float32),
          pltpu.VMEM((_CHUNK, d), jnp.float32),
          pltpu.VMEM_SHARED((_NPAD, d), jnp.float32),
          pltpu.SemaphoreType.DMA,
          pltpu.SemaphoreType.DMA,
          pltpu.SemaphoreType.DMA,
          pltpu.SemaphoreType.DMA,
      ],
  )
  def k(h_hbm, src_hbm, dst_hbm, zeros_hbm, out_hbm, sidx, didxa, didxb,
        bufa, bufb, acc, gsa, gsb, ssa, ssb):
    cid = lax.axis_index("c")
    sid = lax.axis_index("s")
    wid = cid * 16 + sid
    base_r = sid * _RPT
    cbase = wid * _NCHUNK
    pltpu.sync_copy(src_hbm.at[pl.ds(cbase, _NCHUNK)], sidx)
    pltpu.sync_copy(zeros_hbm, bufa)

    def zbody(j, carry):
      pltpu.sync_copy(bufa, acc.at[pl.ds(base_r + j * _CHUNK, _CHUNK)])
      return carry

    lax.fori_loop(0, _RPT // _CHUNK, zbody, 0)
    plsc.subcore_barrier()

    pltpu.sync_copy(dst_hbm.at[cbase], didxa)
    pltpu.sync_copy(dst_hbm.at[cbase + 1], didxb)
    pltpu.async_copy(h_hbm.at[sidx.at[0]], bufa, gsa)
    pltpu.async_copy(h_hbm.at[sidx.at[1]], bufb, gsb)

    def body(i, carry):
      c0 = 2 * i
      pltpu.make_async_copy(h_hbm.at[sidx.at[c0]], bufa, gsa).wait()
      pltpu.async_copy(bufa, acc.at[didxa], ssa, add=True)
      pltpu.make_async_copy(h_hbm.at[sidx.at[c0 + 1]], bufb, gsb).wait()
      pltpu.async_copy(bufb, acc.at[didxb], ssb, add=True)
      pltpu.make_async_copy(bufa, acc.at[didxa], ssa).wait()

      @pl.when(i < _NCHUNK // 2 - 1)
      def _():
        pltpu.sync_copy(dst_hbm.at[cbase + c0 + 2], didxa)
        pltpu.async_copy(h_hbm.at[sidx.at[c0 + 2]], bufa, gsa)

      pltpu.make_async_copy(bufb, acc.at[didxb], ssb).wait()

      @pl.when(i < _NCHUNK // 2 - 1)
      def _():
        pltpu.sync_copy(dst_hbm.at[cbase + c0 + 3], didxb)
        pltpu.async_copy(h_hbm.at[sidx.at[c0 + 3]], bufb, gsb)

      return carry

    lax.fori_loop(0, _NCHUNK // 2, body, 0)
    plsc.subcore_barrier()

    def wbody(j, carry):
      r0 = base_r + j * _CHUNK
      pltpu.sync_copy(acc.at[pl.ds(r0, _CHUNK)], bufa)
      pltpu.sync_copy(bufa, out_hbm.at[cid, pl.ds(r0, _CHUNK)])
      return carry

    lax.fori_loop(0, _RPT // _CHUNK, wbody, 0)

  return k


_BLK = 1024
_GRID = _NPAD // _BLK


def _tc_a(x_ref, w_ref, deg_ref, h_ref, dinv_ref):
  deg = deg_ref[0] + deg_ref[1] + 1.0
  dinv = lax.rsqrt(deg)
  h = jnp.dot(x_ref[...], w_ref[...], preferred_element_type=jnp.float32)
  h_ref[...] = h * dinv[:, None]
  dinv_ref[...] = dinv


def _tc_b(acc_ref, h1_ref, dinv_ref, b1_ref, w2_ref, h2_ref):
  dinv = dinv_ref[...]
  s = acc_ref[0] + acc_ref[1] + h1_ref[...]
  t = jnp.maximum(s * dinv[:, None] + b1_ref[...], 0.0)
  h2 = jnp.dot(t, w2_ref[...], preferred_element_type=jnp.float32)
  h2_ref[...] = h2 * dinv[:, None]


def _tc_c(acc_ref, h2_ref, dinv_ref, b2_ref, o_ref):
  dinv = dinv_ref[...]
  s = acc_ref[0] + acc_ref[1] + h2_ref[...]
  o_ref[...] = s * dinv[:, None] + b2_ref[...]


def _stage_a(x_pad, W1, deg2):
  return pl.pallas_call(
      _tc_a,
      grid=(_GRID,),
      in_specs=[
          pl.BlockSpec((_BLK, 128), lambda i: (i, 0)),
          pl.BlockSpec((128, 128), lambda i: (0, 0)),
          pl.BlockSpec((2, _BLK), lambda i: (0, i)),
      ],
      out_specs=[
          pl.BlockSpec((_BLK, 128), lambda i: (i, 0)),
          pl.BlockSpec((_BLK,), lambda i: (i,)),
      ],
      out_shape=[
          jax.ShapeDtypeStruct((_NPAD, 128), jnp.float32),
          jax.ShapeDtypeStruct((_NPAD,), jnp.float32),
      ],
  )(x_pad, W1, deg2)


def _stage_b(acc1, h1p, dinv, b1, W2):
  return pl.pallas_call(
      _tc_b,
      grid=(_GRID,),
      in_specs=[
          pl.BlockSpec((2, _BLK, 128), lambda i: (0, i, 0)),
          pl.BlockSpec((_BLK, 128), lambda i: (i, 0)),
          pl.BlockSpec((_BLK,), lambda i: (i,)),
          pl.BlockSpec((128,), lambda i: (0,)),
          pl.BlockSpec((128, 64), lambda i: (0, 0)),
      ],
      out_specs=pl.BlockSpec((_BLK, 64), lambda i: (i, 0)),
      out_shape=jax.ShapeDtypeStruct((_NPAD, 64), jnp.float32),
  )(acc1, h1p, dinv, b1, W2)


def _stage_c(acc2, h2p, dinv, b2):
  return pl.pallas_call(
      _tc_c,
      grid=(_GRID,),
      in_specs=[
          pl.BlockSpec((2, _BLK, 64), lambda i: (0, i, 0)),
          pl.BlockSpec((_BLK, 64), lambda i: (i, 0)),
          pl.BlockSpec((_BLK,), lambda i: (i,)),
          pl.BlockSpec((64,), lambda i: (0,)),
      ],
      out_specs=pl.BlockSpec((_BLK, 64), lambda i: (i, 0)),
      out_shape=jax.ShapeDtypeStruct((_NPAD, 64), jnp.float32),
  )(acc2, h2p, dinv, b2)


def kernel(x, edge_index, W1, b1, W2, b2):
  src = edge_index[0].astype(jnp.int32)
  dst = edge_index[1].astype(jnp.int32)
  ppt = (_EPAD - _E) // 32
  padblk = jnp.broadcast_to(_N + jnp.arange(ppt, dtype=jnp.int32), (32, ppt))

  def _pad_edges(a):
    a2 = jnp.concatenate([a.reshape(32, _E // 32), padblk], axis=1)
    return a2.reshape(_EPAD // _CHUNK, _CHUNK)

  src_p = _pad_edges(src)
  dst_p = _pad_edges(dst)

  ones_c = jnp.ones((_CHUNK,), jnp.float32)
  zeros_r = jnp.zeros((_RPT,), jnp.float32)
  zeros128 = jnp.zeros((_CHUNK, 128), jnp.float32)
  zeros64 = jnp.zeros((_CHUNK, 64), jnp.float32)
  x_pad = jnp.zeros((_NPAD, 128), jnp.float32).at[:_N].set(x)

  deg2 = _make_deg_kernel()(dst_p, ones_c, zeros_r)
  h1p, dinv = _stage_a(x_pad, W1, deg2)
  acc1 = _make_scatter_kernel(128)(h1p, src_p, dst_p, zeros128)
  h2p = _stage_b(acc1, h1p, dinv, b1, W2)
  acc2 = _make_scatter_kernel(64)(h2p, src_p, dst_p, zeros64)
  out = _stage_c(acc2, h2p, dinv, b2)
  return out[:_N]


# speedup vs baseline: 2.9331x; 1.1281x over previous
import functools

import jax
import jax.numpy as jnp
from jax import lax
from jax.experimental import pallas as pl
from jax.experimental.pallas import tpu as pltpu
from jax.experimental.pallas import tpu_sc as plsc

_N = 10000
_NPAD = 10240
_RPT = 640
_E = 320000
_CHUNK = 128
_NCHUNK = 80
_EPT = _NCHUNK * _CHUNK
_EPAD = _EPT * 32

@functools.cache
def _make_deg_kernel():
  _mesh = plsc.VectorSubcoreMesh(core_axis_name="c", subcore_axis_name="s")
  @functools.partial(
      pl.kernel, mesh=_mesh,
      out_type=jax.ShapeDtypeStruct((2, _NPAD), jnp.float32),
      scratch_types=[
          pltpu.VMEM((_NCHUNK, _CHUNK), jnp.int32),
          pltpu.VMEM((_CHUNK,), jnp.float32),
          pltpu.VMEM((_RPT,), jnp.float32),
          pltpu.VMEM_SHARED((_NPAD,), jnp.float32),
          pltpu.SemaphoreType.DMA,
      ],
  )
  def k(dst_hbm, ones_hbm, zeros_hbm, out_hbm, didx, onesv, stage, acc, sem):
    cid = lax.axis_index("c")
    sid = lax.axis_index("s")
    wid = cid * 16 + sid
    base_r = sid * _RPT
    pltpu.sync_copy(dst_hbm.at[pl.ds(wid * _NCHUNK, _NCHUNK)], didx)
    pltpu.sync_copy(ones_hbm, onesv)
    pltpu.sync_copy(zeros_hbm, stage)
    pltpu.sync_copy(stage, acc.at[pl.ds(base_r, _RPT)])
    plsc.subcore_barrier()

    def body(i, carry):
      for j in range(8):
        pltpu.async_copy(onesv, acc.at[didx.at[i * 8 + j]], sem, add=True)
      for j in range(8):
        pltpu.make_async_copy(onesv, acc.at[didx.at[i * 8 + j]], sem).wait()
      return carry

    lax.fori_loop(0, _NCHUNK // 8, body, 0)
    plsc.subcore_barrier()
    pltpu.sync_copy(acc.at[pl.ds(base_r, _RPT)], stage)
    pltpu.sync_copy(stage, out_hbm.at[cid, pl.ds(base_r, _RPT)])

  return k


@functools.cache
def _make_scatter_kernel(d):
  _mesh = plsc.VectorSubcoreMesh(core_axis_name="c", subcore_axis_name="s")
  @functools.partial(
      pl.kernel, mesh=_mesh,
      compiler_params=pltpu.CompilerParams(use_tc_tiling_on_sc=(d == 128)),
      out_type=jax.ShapeDtypeStruct((2, _NPAD, d), jnp.float32),
      scratch_types=[
          pltpu.VMEM((_NCHUNK, _CHUNK), jnp.int32),
          pltpu.VMEM((_CHUNK,), jnp.int32),
          pltpu.VMEM((_CHUNK,), jnp.int32),
          pltpu.VMEM((_CHUNK, d), jnp.float32),
          pltpu.VMEM((_CHUNK, d), jnp.float32),
          pltpu.VMEM_SHARED((_NPAD, d), jnp.float32),
          pltpu.SemaphoreType.DMA,
          pltpu.SemaphoreType.DMA,
      ],
  )
  def k(h_hbm, src_hbm, dst_hbm, zeros_hbm, out_hbm, sidx, didxa, didxb,
        bufa, bufb, acc, sema, semb):
    cid = lax.axis_index("c")
    sid = lax.axis_index("s")
    wid = cid * 16 + sid
    base_r = sid * _RPT
    cbase = wid * _NCHUNK
    pltpu.sync_copy(src_hbm.at[pl.ds(cbase, _NCHUNK)], sidx)
    pltpu.sync_copy(zeros_hbm, bufa)

    def zbody(j, carry):
      pltpu.sync_copy(bufa, acc.at[pl.ds(base_r + j * _CHUNK, _CHUNK)])
      return carry

    lax.fori_loop(0, _RPT // _CHUNK, zbody, 0)
    plsc.subcore_barrier()

    pltpu.async_copy(h_hbm.at[sidx.at[0]], bufa, sema)
    pltpu.sync_copy(dst_hbm.at[cbase], didxa)

    def body(i, carry):
      c0 = 2 * i
      pltpu.async_copy(h_hbm.at[sidx.at[c0 + 1]], bufb, semb)
      pltpu.sync_copy(dst_hbm.at[cbase + c0 + 1], didxb)
      pltpu.make_async_copy(h_hbm.at[sidx.at[c0]], bufa, sema).wait()
      pltpu.sync_copy(bufa, acc.at[didxa], add=True)

      @pl.when(i < _NCHUNK // 2 - 1)
      def _():
        pltpu.async_copy(h_hbm.at[sidx.at[c0 + 2]], bufa, sema)
        pltpu.sync_copy(dst_hbm.at[cbase + c0 + 2], didxa)

      pltpu.make_async_copy(h_hbm.at[sidx.at[c0 + 1]], bufb, semb).wait()
      pltpu.sync_copy(bufb, acc.at[didxb], add=True)
      return carry

    lax.fori_loop(0, _NCHUNK // 2, body, 0)
    plsc.subcore_barrier()

    def wbody(j, carry):
      r0 = base_r + j * _CHUNK
      pltpu.sync_copy(acc.at[pl.ds(r0, _CHUNK)], bufa)
      pltpu.sync_copy(bufa, out_hbm.at[cid, pl.ds(r0, _CHUNK)])
      return carry

    lax.fori_loop(0, _RPT // _CHUNK, wbody, 0)

  return k


_BLK = 1024
_GRID = _NPAD // _BLK


def _tc_a(x_ref, w_ref, deg_ref, h_ref, dinv_ref):
  deg = deg_ref[0] + deg_ref[1] + 1.0
  dinv = lax.rsqrt(deg)
  h = jnp.dot(x_ref[...], w_ref[...], preferred_element_type=jnp.float32)
  h_ref[...] = h * dinv[:, None]
  dinv_ref[...] = dinv


def _tc_b(acc_ref, h1_ref, dinv_ref, b1_ref, w2_ref, h2_ref):
  dinv = dinv_ref[...]
  s = acc_ref[0] + acc_ref[1] + h1_ref[...]
  t = jnp.maximum(s * dinv[:, None] + b1_ref[...], 0.0)
  h2 = jnp.dot(t, w2_ref[...], preferred_element_type=jnp.float32)
  h2_ref[...] = h2 * dinv[:, None]


def _tc_c(acc_ref, h2_ref, dinv_ref, b2_ref, o_ref):
  dinv = dinv_ref[...]
  s = acc_ref[0] + acc_ref[1] + h2_ref[...]
  o_ref[...] = s * dinv[:, None] + b2_ref[...]


def _stage_a(x_pad, W1, deg2):
  return pl.pallas_call(
      _tc_a,
      grid=(_GRID,),
      in_specs=[
          pl.BlockSpec((_BLK, 128), lambda i: (i, 0)),
          pl.BlockSpec((128, 128), lambda i: (0, 0)),
          pl.BlockSpec((2, _BLK), lambda i: (0, i)),
      ],
      out_specs=[
          pl.BlockSpec((_BLK, 128), lambda i: (i, 0)),
          pl.BlockSpec((_BLK,), lambda i: (i,)),
      ],
      out_shape=[
          jax.ShapeDtypeStruct((_NPAD, 128), jnp.float32),
          jax.ShapeDtypeStruct((_NPAD,), jnp.float32),
      ],
  )(x_pad, W1, deg2)


def _stage_b(acc1, h1p, dinv, b1, W2):
  return pl.pallas_call(
      _tc_b,
      grid=(_GRID,),
      in_specs=[
          pl.BlockSpec((2, _BLK, 128), lambda i: (0, i, 0)),
          pl.BlockSpec((_BLK, 128), lambda i: (i, 0)),
          pl.BlockSpec((_BLK,), lambda i: (i,)),
          pl.BlockSpec((128,), lambda i: (0,)),
          pl.BlockSpec((128, 64), lambda i: (0, 0)),
      ],
      out_specs=pl.BlockSpec((_BLK, 64), lambda i: (i, 0)),
      out_shape=jax.ShapeDtypeStruct((_NPAD, 64), jnp.float32),
  )(acc1, h1p, dinv, b1, W2)


def _stage_c(acc2, h2p, dinv, b2):
  return pl.pallas_call(
      _tc_c,
      grid=(_GRID,),
      in_specs=[
          pl.BlockSpec((2, _BLK, 64), lambda i: (0, i, 0)),
          pl.BlockSpec((_BLK, 64), lambda i: (i, 0)),
          pl.BlockSpec((_BLK,), lambda i: (i,)),
          pl.BlockSpec((64,), lambda i: (0,)),
      ],
      out_specs=pl.BlockSpec((_BLK, 64), lambda i: (i, 0)),
      out_shape=jax.ShapeDtypeStruct((_NPAD, 64), jnp.float32),
  )(acc2, h2p, dinv, b2)


def kernel(x, edge_index, W1, b1, W2, b2):
  src = edge_index[0].astype(jnp.int32)
  dst = edge_index[1].astype(jnp.int32)
  ppt = (_EPAD - _E) // 32
  padblk = jnp.broadcast_to(_N + jnp.arange(ppt, dtype=jnp.int32), (32, ppt))

  def _pad_edges(a):
    a2 = jnp.concatenate([a.reshape(32, _E // 32), padblk], axis=1)
    return a2.reshape(_EPAD // _CHUNK, _CHUNK)

  src_p = _pad_edges(src)
  dst_p = _pad_edges(dst)

  ones_c = jnp.ones((_CHUNK,), jnp.float32)
  zeros_r = jnp.zeros((_RPT,), jnp.float32)
  zeros128 = jnp.zeros((_CHUNK, 128), jnp.float32)
  zeros64 = jnp.zeros((_CHUNK, 64), jnp.float32)
  x_pad = jnp.zeros((_NPAD, 128), jnp.float32).at[:_N].set(x)

  deg2 = _make_deg_kernel()(dst_p, ones_c, zeros_r)
  h1p, dinv = _stage_a(x_pad, W1, deg2)
  acc1 = _make_scatter_kernel(128)(h1p, src_p, dst_p, zeros128)
  h2p = _stage_b(acc1, h1p, dinv, b1, W2)
  acc2 = _make_scatter_kernel(64)(h2p, src_p, dst_p, zeros64)
  out = _stage_c(acc2, h2p, dinv, b2)
  return out[:_N]
